# baseline (speedup 1.0000x reference)
import functools
import math

import jax
import jax.numpy as jnp
from jax import lax
from jax.experimental import pallas as pl
from jax.experimental.pallas import tpu as pltpu
from jax.experimental.pallas import tpu_sc as plsc

DIM = 64
N_ROUNDS = 8
L_SIZE = 100000
C_SIZE = 40000

NC = 2
NS = 16
EB = 128
KB = 4
SUPER = EB * KB
SWEEP = NS * SUPER
ZROWS = 256


def _make_sc_aggregate(e_rows, n_iter, n_src, s_out, G, Cg):
    s_pad = -(-(s_out + 1) // (NS * 8)) * (NS * 8)
    stripe_z = s_pad // NS
    stripe_o = (s_out // (NS * 8)) * 8
    o_rem = s_out - NS * stripe_o
    n_zfull = stripe_z // ZROWS
    z_rem = stripe_z - n_zfull * ZROWS
    mesh = plsc.VectorSubcoreMesh(
        core_axis_name="c", subcore_axis_name="s",
        num_cores=NC, num_subcores=NS)

    out_type = tuple(jax.ShapeDtypeStruct((s_out, Cg), jnp.float32)
                     for _ in range(G))
    scratch = [
        pltpu.VMEM_SHARED((s_pad, Cg), jnp.float32),
        pltpu.VMEM((KB, EB), jnp.int32),
        pltpu.VMEM((KB, EB), jnp.int32),
        pltpu.VMEM((KB, EB), jnp.int32),
        pltpu.VMEM((KB, EB), jnp.int32),
        pltpu.VMEM((KB * EB, Cg), jnp.float32),
        pltpu.VMEM((KB * EB, Cg), jnp.float32),
        pltpu.VMEM((ZROWS, Cg), jnp.float32),
        pltpu.SemaphoreType.DMA,
        pltpu.SemaphoreType.DMA,
        pltpu.SemaphoreType.DMA,
        pltpu.SemaphoreType.DMA,
    ]

    def body(src2d, dst2d, *rest):
        msgs = rest[:G]
        zhbm = rest[G]
        outs = rest[G + 1:2 * G + 1]
        (acc, sidx0, didx0, sidx1, didx1, rows0, rows1, zbuf,
         semg0, semg1, sems0, sems1) = rest[2 * G + 1:]
        cid = lax.axis_index("c")
        sid = lax.axis_index("s")
        pltpu.sync_copy(zhbm, zbuf)

        for g in range(G):
            @pl.when(cid == (g % NC))
            def _(g=g):
                zb = sid * stripe_z
                for t in range(n_zfull):
                    pltpu.sync_copy(zbuf, acc.at[pl.ds(zb + t * ZROWS, ZROWS)])
                if z_rem:
                    pltpu.sync_copy(zbuf.at[pl.ds(0, z_rem)],
                                    acc.at[pl.ds(zb + n_zfull * ZROWS, z_rem)])
                plsc.subcore_barrier()

                def load(i, sI, dI):
                    r0 = lax.min((sid + i * NS) * KB, e_rows - KB)
                    pltpu.sync_copy(src2d.at[pl.ds(r0, KB)], sI)
                    pltpu.sync_copy(dst2d.at[pl.ds(r0, KB)], dI)

                def fire_g(sI, rws, sem):
                    for k in range(KB):
                        pltpu.async_copy(msgs[g].at[sI.at[k]],
                                         rws.at[pl.ds(k * EB, EB)], sem)

                def wait_g(sI, rws, sem):
                    for k in range(KB):
                        pltpu.make_async_copy(
                            msgs[g].at[sI.at[k]],
                            rws.at[pl.ds(k * EB, EB)], sem).wait()

                def fire_s(dI, rws, sem):
                    for k in range(KB):
                        pltpu.async_copy(rws.at[pl.ds(k * EB, EB)],
                                         acc.at[dI.at[k]], sem, add=True)

                def drain_s(dI, rws, sem):
                    for k in range(KB):
                        pltpu.make_async_copy(
                            rws.at[pl.ds(k * EB, EB)],
                            acc.at[dI.at[k]], sem).wait()

                def pair(j, first):
                    wait_g(sidx0, rows0, semg0)
                    if not first:
                        drain_s(didx1, rows1, sems1)
                    load(2 * j + 1, sidx1, didx1)
                    fire_s(didx0, rows0, sems0)
                    fire_g(sidx1, rows1, semg1)
                    wait_g(sidx1, rows1, semg1)
                    drain_s(didx0, rows0, sems0)
                    load(2 * j + 2, sidx0, didx0)
                    fire_s(didx1, rows1, sems1)
                    fire_g(sidx0, rows0, semg0)

                load(0, sidx0, didx0)
                fire_g(sidx0, rows0, semg0)
                pair(0, True)
                lax.fori_loop(1, n_iter // 2,
                              lambda j, c: (pair(j, False), c)[1], 0)
                wait_g(sidx0, rows0, semg0)
                drain_s(didx1, rows1, sems1)
                plsc.subcore_barrier()

                cb = sid * stripe_o
                pltpu.sync_copy(acc.at[pl.ds(cb, stripe_o)],
                                outs[g].at[pl.ds(cb, stripe_o)])
                if o_rem:
                    @pl.when(sid == 0)
                    def _():
                        pltpu.sync_copy(
                            acc.at[pl.ds(NS * stripe_o, o_rem)],
                            outs[g].at[pl.ds(NS * stripe_o, o_rem)])
                plsc.subcore_barrier()

    return pl.kernel(body, out_type=out_type, mesh=mesh,
                     scratch_types=scratch,
                     compiler_params=pltpu.CompilerParams(
                         use_tc_tiling_on_sc=False))


def _ln(x, g, b, eps=1e-5):
    mu = jnp.mean(x, axis=1, keepdims=True)
    var = jnp.mean((x - mu) ** 2, axis=1, keepdims=True)
    return (x - mu) * lax.rsqrt(var + eps) * g + b


def _mlp3(x, W0, b0, W1, b1, W2, b2):
    h = jnp.maximum(jnp.dot(x, W0, preferred_element_type=jnp.float32) + b0, 0.0)
    h = jnp.maximum(jnp.dot(h, W1, preferred_element_type=jnp.float32) + b1, 0.0)
    return jnp.dot(h, W2, preferred_element_type=jnp.float32) + b2


def _swap_pairs(x):
    r = x.shape[0]
    nxt = jnp.concatenate([x[1:], x[:1]], axis=0)
    prv = jnp.concatenate([x[-1:], x[:-1]], axis=0)
    row = lax.broadcasted_iota(jnp.int32, (r, 1), 0)
    return jnp.where(row % 2 == 0, nxt, prv)


def _sigmoid(x):
    return 1.0 / (1.0 + jnp.exp(-x))


def _lstm_block(x, h, c, W_ih, W_hh, g_ih, b_ih, g_hh, b_hh, g_c, b_c):
    gi = _ln(jnp.dot(x, W_ih, preferred_element_type=jnp.float32), g_ih, b_ih)
    gh = _ln(jnp.dot(h, W_hh, preferred_element_type=jnp.float32), g_hh, b_hh)
    gates = gi + gh
    i_g = gates[:, 0:DIM]
    f_g = gates[:, DIM:2 * DIM]
    g_g = gates[:, 2 * DIM:3 * DIM]
    o_g = gates[:, 3 * DIM:4 * DIM]
    c_new = _sigmoid(f_g) * c + _sigmoid(i_g) * jnp.tanh(g_g)
    h_new = _sigmoid(o_g) * jnp.tanh(_ln(c_new, g_c, b_c))
    return h_new, c_new


_FULL2 = lambda shape: pl.BlockSpec(shape, lambda i: (0, 0))


def _make_lstm_c(R):
    nb = C_SIZE // R

    def body(a0, a1, h, c, W_ih, W_hh, g_ih, b_ih, g_hh, b_hh, g_c, b_c,
             W0, b0, W1, b1, W2, b2, oh, oc, m0, m1, m2, m3):
        x = jnp.concatenate([a0[...], a1[...]], axis=1)
        h_new, c_new = _lstm_block(
            x, h[...], c[...], W_ih[...], W_hh[...], g_ih[...], b_ih[...],
            g_hh[...], b_hh[...], g_c[...], b_c[...])
        oh[...] = h_new
        oc[...] = c_new
        m = _mlp3(h_new, W0[...], b0[...], W1[...], b1[...], W2[...], b2[...])
        m0[...] = m[:, 0:16]
        m1[...] = m[:, 16:32]
        m2[...] = m[:, 32:48]
        m3[...] = m[:, 48:64]

    row = lambda shape: pl.BlockSpec(shape, lambda i: (i, 0))
    in_specs = [row((R, 32)), row((R, 32)), row((R, DIM)), row((R, DIM)),
                _FULL2((DIM, 4 * DIM)), _FULL2((DIM, 4 * DIM)),
                _FULL2((1, 4 * DIM)), _FULL2((1, 4 * DIM)),
                _FULL2((1, 4 * DIM)), _FULL2((1, 4 * DIM)),
                _FULL2((1, DIM)), _FULL2((1, DIM)),
                _FULL2((DIM, DIM)), _FULL2((1, DIM)),
                _FULL2((DIM, DIM)), _FULL2((1, DIM)),
                _FULL2((DIM, DIM)), _FULL2((1, DIM))]
    out_specs = [row((R, DIM)), row((R, DIM)),
                 row((R, 16)), row((R, 16)), row((R, 16)), row((R, 16))]
    out_shape = [jax.ShapeDtypeStruct((C_SIZE, DIM), jnp.float32),
                 jax.ShapeDtypeStruct((C_SIZE, DIM), jnp.float32)] + \
                [jax.ShapeDtypeStruct((C_SIZE, 16), jnp.float32)] * 4
    return pl.pallas_call(body, grid=(nb,), in_specs=in_specs,
                          out_specs=out_specs, out_shape=out_shape)


def _make_lstm_l(R):
    nb = L_SIZE // R

    def body(a0, a1, a2, a3, h, c, W_ih, W_hh, g_ih, b_ih, g_hh, b_hh,
             g_c, b_c, W0, b0, W1, b1, W2, b2, oh, oc, m0, m1):
        hv = h[...]
        x = jnp.concatenate([a0[...], a1[...], a2[...], a3[...],
                             _swap_pairs(hv)], axis=1)
        h_new, c_new = _lstm_block(
            x, hv, c[...], W_ih[...], W_hh[...], g_ih[...], b_ih[...],
            g_hh[...], b_hh[...], g_c[...], b_c[...])
        oh[...] = h_new
        oc[...] = c_new
        m = _mlp3(h_new, W0[...], b0[...], W1[...], b1[...], W2[...], b2[...])
        m0[...] = m[:, 0:32]
        m1[...] = m[:, 32:64]

    row = lambda shape: pl.BlockSpec(shape, lambda i: (i, 0))
    in_specs = [row((R, 16))] * 4 + [row((R, DIM)), row((R, DIM)),
                _FULL2((2 * DIM, 4 * DIM)), _FULL2((DIM, 4 * DIM)),
                _FULL2((1, 4 * DIM)), _FULL2((1, 4 * DIM)),
                _FULL2((1, 4 * DIM)), _FULL2((1, 4 * DIM)),
                _FULL2((1, DIM)), _FULL2((1, DIM)),
                _FULL2((DIM, DIM)), _FULL2((1, DIM)),
                _FULL2((DIM, DIM)), _FULL2((1, DIM)),
                _FULL2((DIM, DIM)), _FULL2((1, DIM))]
    out_specs = [row((R, DIM)), row((R, DIM)), row((R, 32)), row((R, 32))]
    out_shape = [jax.ShapeDtypeStruct((L_SIZE, DIM), jnp.float32),
                 jax.ShapeDtypeStruct((L_SIZE, DIM), jnp.float32),
                 jax.ShapeDtypeStruct((L_SIZE, 32), jnp.float32),
                 jax.ShapeDtypeStruct((L_SIZE, 32), jnp.float32)]
    return pl.pallas_call(body, grid=(nb,), in_specs=in_specs,
                          out_specs=out_specs, out_shape=out_shape)


def _init_msg(row64, W0, b0, W1, b1, W2, b2):
    def body(x, W0r, b0r, W1r, b1r, W2r, b2r, o):
        o[...] = _mlp3(x[...], W0r[...], b0r[...], W1r[...], b1r[...],
                       W2r[...], b2r[...])
    f = pl.pallas_call(
        body,
        out_shape=jax.ShapeDtypeStruct((8, DIM), jnp.float32))
    return f(jnp.broadcast_to(row64, (8, DIM)), W0, b0, W1, b1, W2, b2)


def _make_readout(R):
    nb = L_SIZE // R

    def body(h, W0, b0, W1, b1, W2, b2, o):
        s = _mlp3(h[...], W0[...], b0[...], W1[...], b1[...], W2[...], b2[...])
        sp = _swap_pairs(s)
        m = jnp.maximum(s, sp)
        e1 = jnp.exp(s - m)
        e2 = jnp.exp(sp - m)
        o[...] = e1 / (e1 + e2)

    row = lambda shape: pl.BlockSpec(shape, lambda i: (i, 0))
    in_specs = [row((R, DIM)),
                _FULL2((DIM, DIM)), _FULL2((1, DIM)),
                _FULL2((DIM, DIM)), _FULL2((1, DIM)),
                _FULL2((DIM, 1)), _FULL2((1, 1))]
    return pl.pallas_call(
        body, grid=(nb,), in_specs=in_specs, out_specs=row((R, 1)),
        out_shape=jax.ShapeDtypeStruct((L_SIZE, 1), jnp.float32))


def _prep_edges(idx, pad, fill):
    p = jnp.concatenate([idx, jnp.full((pad,), fill, jnp.int32)])
    return p.reshape(-1, EB)


def kernel(l_edge_index, c_edge_index, l_size, c_size, params):
    E = l_edge_index.shape[0]
    n_iter = -(-E // SWEEP)
    n_iter += n_iter % 2
    e_pad = n_iter * SWEEP
    e_rows = e_pad // EB

    src_l = _prep_edges(l_edge_index, e_pad - E, 0)
    dst_c = _prep_edges(c_edge_index, e_pad - E, C_SIZE)
    src_c = _prep_edges(c_edge_index, e_pad - E, 0)
    dst_l = _prep_edges(l_edge_index, e_pad - E, L_SIZE)

    sc_l2c = _make_sc_aggregate(e_rows, n_iter, L_SIZE, C_SIZE, 2, 32)
    sc_c2l = _make_sc_aggregate(e_rows, n_iter, C_SIZE, L_SIZE, 4, 16)
    lstm_c = _make_lstm_c(2000)
    lstm_l = _make_lstm_l(2000)
    readout = _make_readout(2000)

    p = params
    cc, lc = p['c_cell'], p['l_cell']
    r2 = lambda v: v.reshape(1, -1)
    denom = math.sqrt(DIM)

    l_hidden = jnp.broadcast_to(p['l_init'] / denom, (L_SIZE, DIM))
    c_hidden = jnp.broadcast_to(p['c_init'] / denom, (C_SIZE, DIM))
    l_state = jnp.zeros((L_SIZE, DIM), jnp.float32)
    c_state = jnp.zeros((C_SIZE, DIM), jnp.float32)

    z32 = jnp.zeros((ZROWS, 32), jnp.float32)
    z16 = jnp.zeros((ZROWS, 16), jnp.float32)

    mrow = _init_msg(p['l_init'] / denom, p['l2c_W'][0], r2(p['l2c_b'][0]),
                     p['l2c_W'][1], r2(p['l2c_b'][1]),
                     p['l2c_W'][2], r2(p['l2c_b'][2]))[0]
    msg_l = (jnp.broadcast_to(mrow[0:32], (L_SIZE, 32)),
             jnp.broadcast_to(mrow[32:64], (L_SIZE, 32)))

    for _ in range(N_ROUNDS):
        la0, la1 = sc_l2c(src_l, dst_c, msg_l[0], msg_l[1], z32)
        c_hidden, c_state, cm0, cm1, cm2, cm3 = lstm_c(
            la0, la1, c_hidden, c_state,
            cc['W_ih'], cc['W_hh'], r2(cc['g_ih']), r2(cc['b_ih']),
            r2(cc['g_hh']), r2(cc['b_hh']), r2(cc['g_c']), r2(cc['b_c']),
            p['c2l_W'][0], r2(p['c2l_b'][0]), p['c2l_W'][1], r2(p['c2l_b'][1]),
            p['c2l_W'][2], r2(p['c2l_b'][2]))
        ca0, ca1, ca2, ca3 = sc_c2l(src_c, dst_l, cm0, cm1, cm2, cm3, z16)
        l_hidden, l_state, ml0, ml1 = lstm_l(
            ca0, ca1, ca2, ca3, l_hidden, l_state,
            lc['W_ih'], lc['W_hh'], r2(lc['g_ih']), r2(lc['b_ih']),
            r2(lc['g_hh']), r2(lc['b_hh']), r2(lc['g_c']), r2(lc['b_c']),
            p['l2c_W'][0], r2(p['l2c_b'][0]), p['l2c_W'][1], r2(p['l2c_b'][1]),
            p['l2c_W'][2], r2(p['l2c_b'][2]))
        msg_l = (ml0, ml1)

    probs = readout(l_hidden, p['ro_W'][0], r2(p['ro_b'][0]),
                    p['ro_W'][1], r2(p['ro_b'][1]),
                    p['ro_W'][2], r2(p['ro_b'][2]))
    return probs.reshape(-1, 2)


# speedup vs baseline: 1.1798x; 1.1798x over previous
import functools
import math

import jax
import jax.numpy as jnp
from jax import lax
from jax.experimental import pallas as pl
from jax.experimental.pallas import tpu as pltpu
from jax.experimental.pallas import tpu_sc as plsc

DIM = 64
N_ROUNDS = 8
L_SIZE = 100000
C_SIZE = 40000

NC = 2
NS = 16
EB = 128
KB = 10
SUPER = EB * KB
SWEEP = NS * SUPER
ZROWS = 128


def _make_sc_aggregate(e_rows, n_iter, n_src, s_out, G, Cg):
    s_pad = -(-(s_out + 1) // (NS * 8)) * (NS * 8)
    stripe_z = s_pad // NS
    stripe_o = (s_out // (NS * 8)) * 8
    o_rem = s_out - NS * stripe_o
    n_zfull = stripe_z // ZROWS
    z_rem = stripe_z - n_zfull * ZROWS
    mesh = plsc.VectorSubcoreMesh(
        core_axis_name="c", subcore_axis_name="s",
        num_cores=NC, num_subcores=NS)

    out_type = tuple(jax.ShapeDtypeStruct((s_out, Cg), jnp.float32)
                     for _ in range(G))
    scratch = [
        pltpu.VMEM_SHARED((s_pad, Cg), jnp.float32),
        pltpu.VMEM((KB, EB), jnp.int32),
        pltpu.VMEM((KB, EB), jnp.int32),
        pltpu.VMEM((KB, EB), jnp.int32),
        pltpu.VMEM((KB, EB), jnp.int32),
        pltpu.VMEM((KB * EB, Cg), jnp.float32),
        pltpu.VMEM((ZROWS, Cg), jnp.float32),
        pltpu.SemaphoreType.DMA,
        pltpu.SemaphoreType.DMA,
        pltpu.SemaphoreType.DMA,
    ]

    def body(src2d, dst2d, *rest):
        msgs = rest[:G]
        zhbm = rest[G]
        outs = rest[G + 1:2 * G + 1]
        (acc, sidx0, didx0, sidx1, didx1, rows, zbuf,
         semi, semg, sems) = rest[2 * G + 1:]
        cid = lax.axis_index("c")
        sid = lax.axis_index("s")
        pltpu.sync_copy(zhbm, zbuf)

        for g in range(G):
            @pl.when(cid == (g % NC))
            def _(g=g):
                zb = sid * stripe_z
                for t in range(n_zfull):
                    pltpu.sync_copy(zbuf, acc.at[pl.ds(zb + t * ZROWS,
                                                       ZROWS)])
                if z_rem:
                    pltpu.sync_copy(zbuf.at[pl.ds(0, z_rem)],
                                    acc.at[pl.ds(zb + n_zfull * ZROWS,
                                                 z_rem)])
                plsc.subcore_barrier()

                def half(i, sP, dP, sN, dN):
                    r0n = lax.min((sid + (i + 1) * NS) * KB, e_rows - KB)
                    pltpu.async_copy(src2d.at[pl.ds(r0n, KB)], sN, semi)
                    pltpu.async_copy(dst2d.at[pl.ds(r0n, KB)], dN, semi)
                    for k in range(KB):
                        pltpu.async_copy(msgs[g].at[sP.at[k]],
                                         rows.at[pl.ds(k * EB, EB)], semg)
                    pltpu.make_async_copy(msgs[g].at[pl.ds(0, KB * EB)],
                                         rows, semg).wait()
                    for k in range(KB):
                        pltpu.async_copy(rows.at[pl.ds(k * EB, EB)],
                                         acc.at[dP.at[k]], sems, add=True)
                    pltpu.make_async_copy(rows, acc.at[pl.ds(0, KB * EB)],
                                         sems).wait()
                    pltpu.make_async_copy(src2d.at[pl.ds(r0n, KB)], sN,
                                         semi).wait()
                    pltpu.make_async_copy(dst2d.at[pl.ds(r0n, KB)], dN,
                                         semi).wait()

                def pair(j, c):
                    half(2 * j, sidx0, didx0, sidx1, didx1)
                    half(2 * j + 1, sidx1, didx1, sidx0, didx0)
                    return c

                r00 = (sid * KB)
                pltpu.sync_copy(src2d.at[pl.ds(r00, KB)], sidx0)
                pltpu.sync_copy(dst2d.at[pl.ds(r00, KB)], didx0)
                lax.fori_loop(0, n_iter // 2, pair, 0)
                plsc.subcore_barrier()

                cb = sid * stripe_o
                pltpu.sync_copy(acc.at[pl.ds(cb, stripe_o)],
                                outs[g].at[pl.ds(cb, stripe_o)])
                if o_rem:
                    @pl.when(sid == 0)
                    def _():
                        pltpu.sync_copy(
                            acc.at[pl.ds(NS * stripe_o, o_rem)],
                            outs[g].at[pl.ds(NS * stripe_o, o_rem)])
                plsc.subcore_barrier()

    return pl.kernel(body, out_type=out_type, mesh=mesh,
                     scratch_types=scratch,
                     compiler_params=pltpu.CompilerParams(
                         use_tc_tiling_on_sc=False))


def _ln(x, g, b, eps=1e-5):
    mu = jnp.mean(x, axis=1, keepdims=True)
    var = jnp.mean((x - mu) ** 2, axis=1, keepdims=True)
    return (x - mu) * lax.rsqrt(var + eps) * g + b


def _mlp3(x, W0, b0, W1, b1, W2, b2):
    h = jnp.maximum(jnp.dot(x, W0, preferred_element_type=jnp.float32) + b0, 0.0)
    h = jnp.maximum(jnp.dot(h, W1, preferred_element_type=jnp.float32) + b1, 0.0)
    return jnp.dot(h, W2, preferred_element_type=jnp.float32) + b2


def _swap_pairs(x):
    r = x.shape[0]
    nxt = jnp.concatenate([x[1:], x[:1]], axis=0)
    prv = jnp.concatenate([x[-1:], x[:-1]], axis=0)
    row = lax.broadcasted_iota(jnp.int32, (r, 1), 0)
    return jnp.where(row % 2 == 0, nxt, prv)


def _sigmoid(x):
    return 1.0 / (1.0 + jnp.exp(-x))


def _lstm_block(x, h, c, W_ih, W_hh, g_ih, b_ih, g_hh, b_hh, g_c, b_c):
    gi = _ln(jnp.dot(x, W_ih, preferred_element_type=jnp.float32), g_ih, b_ih)
    gh = _ln(jnp.dot(h, W_hh, preferred_element_type=jnp.float32), g_hh, b_hh)
    gates = gi + gh
    i_g = gates[:, 0:DIM]
    f_g = gates[:, DIM:2 * DIM]
    g_g = gates[:, 2 * DIM:3 * DIM]
    o_g = gates[:, 3 * DIM:4 * DIM]
    c_new = _sigmoid(f_g) * c + _sigmoid(i_g) * jnp.tanh(g_g)
    h_new = _sigmoid(o_g) * jnp.tanh(_ln(c_new, g_c, b_c))
    return h_new, c_new


_FULL2 = lambda shape: pl.BlockSpec(shape, lambda i: (0, 0))


def _make_lstm_c(R):
    nb = C_SIZE // R

    def body(a0, a1, h, c, W_ih, W_hh, g_ih, b_ih, g_hh, b_hh, g_c, b_c,
             W0, b0, W1, b1, W2, b2, oh, oc, m0, m1, m2, m3):
        x = jnp.concatenate([a0[...], a1[...]], axis=1)
        h_new, c_new = _lstm_block(
            x, h[...], c[...], W_ih[...], W_hh[...], g_ih[...], b_ih[...],
            g_hh[...], b_hh[...], g_c[...], b_c[...])
        oh[...] = h_new
        oc[...] = c_new
        m = _mlp3(h_new, W0[...], b0[...], W1[...], b1[...], W2[...], b2[...])
        m0[...] = m[:, 0:16]
        m1[...] = m[:, 16:32]
        m2[...] = m[:, 32:48]
        m3[...] = m[:, 48:64]

    row = lambda shape: pl.BlockSpec(shape, lambda i: (i, 0))
    in_specs = [row((R, 32)), row((R, 32)), row((R, DIM)), row((R, DIM)),
                _FULL2((DIM, 4 * DIM)), _FULL2((DIM, 4 * DIM)),
                _FULL2((1, 4 * DIM)), _FULL2((1, 4 * DIM)),
                _FULL2((1, 4 * DIM)), _FULL2((1, 4 * DIM)),
                _FULL2((1, DIM)), _FULL2((1, DIM)),
                _FULL2((DIM, DIM)), _FULL2((1, DIM)),
                _FULL2((DIM, DIM)), _FULL2((1, DIM)),
                _FULL2((DIM, DIM)), _FULL2((1, DIM))]
    out_specs = [row((R, DIM)), row((R, DIM)),
                 row((R, 16)), row((R, 16)), row((R, 16)), row((R, 16))]
    out_shape = [jax.ShapeDtypeStruct((C_SIZE, DIM), jnp.float32),
                 jax.ShapeDtypeStruct((C_SIZE, DIM), jnp.float32)] + \
                [jax.ShapeDtypeStruct((C_SIZE, 16), jnp.float32)] * 4
    return pl.pallas_call(body, grid=(nb,), in_specs=in_specs,
                          out_specs=out_specs, out_shape=out_shape)


def _make_lstm_l(R):
    nb = L_SIZE // R

    def body(a0, a1, a2, a3, h, c, W_ih, W_hh, g_ih, b_ih, g_hh, b_hh,
             g_c, b_c, W0, b0, W1, b1, W2, b2, oh, oc, m0, m1):
        hv = h[...]
        x = jnp.concatenate([a0[...], a1[...], a2[...], a3[...],
                             _swap_pairs(hv)], axis=1)
        h_new, c_new = _lstm_block(
            x, hv, c[...], W_ih[...], W_hh[...], g_ih[...], b_ih[...],
            g_hh[...], b_hh[...], g_c[...], b_c[...])
        oh[...] = h_new
        oc[...] = c_new
        m = _mlp3(h_new, W0[...], b0[...], W1[...], b1[...], W2[...], b2[...])
        m0[...] = m[:, 0:32]
        m1[...] = m[:, 32:64]

    row = lambda shape: pl.BlockSpec(shape, lambda i: (i, 0))
    in_specs = [row((R, 16))] * 4 + [row((R, DIM)), row((R, DIM)),
                _FULL2((2 * DIM, 4 * DIM)), _FULL2((DIM, 4 * DIM)),
                _FULL2((1, 4 * DIM)), _FULL2((1, 4 * DIM)),
                _FULL2((1, 4 * DIM)), _FULL2((1, 4 * DIM)),
                _FULL2((1, DIM)), _FULL2((1, DIM)),
                _FULL2((DIM, DIM)), _FULL2((1, DIM)),
                _FULL2((DIM, DIM)), _FULL2((1, DIM)),
                _FULL2((DIM, DIM)), _FULL2((1, DIM))]
    out_specs = [row((R, DIM)), row((R, DIM)), row((R, 32)), row((R, 32))]
    out_shape = [jax.ShapeDtypeStruct((L_SIZE, DIM), jnp.float32),
                 jax.ShapeDtypeStruct((L_SIZE, DIM), jnp.float32),
                 jax.ShapeDtypeStruct((L_SIZE, 32), jnp.float32),
                 jax.ShapeDtypeStruct((L_SIZE, 32), jnp.float32)]
    return pl.pallas_call(body, grid=(nb,), in_specs=in_specs,
                          out_specs=out_specs, out_shape=out_shape)


def _init_msg(row64, W0, b0, W1, b1, W2, b2):
    def body(x, W0r, b0r, W1r, b1r, W2r, b2r, o):
        o[...] = _mlp3(x[...], W0r[...], b0r[...], W1r[...], b1r[...],
                       W2r[...], b2r[...])
    f = pl.pallas_call(
        body,
        out_shape=jax.ShapeDtypeStruct((8, DIM), jnp.float32))
    return f(jnp.broadcast_to(row64, (8, DIM)), W0, b0, W1, b1, W2, b2)


def _make_readout(R):
    nb = L_SIZE // R

    def body(h, W0, b0, W1, b1, W2, b2, o):
        s = _mlp3(h[...], W0[...], b0[...], W1[...], b1[...], W2[...], b2[...])
        sp = _swap_pairs(s)
        m = jnp.maximum(s, sp)
        e1 = jnp.exp(s - m)
        e2 = jnp.exp(sp - m)
        o[...] = e1 / (e1 + e2)

    row = lambda shape: pl.BlockSpec(shape, lambda i: (i, 0))
    in_specs = [row((R, DIM)),
                _FULL2((DIM, DIM)), _FULL2((1, DIM)),
                _FULL2((DIM, DIM)), _FULL2((1, DIM)),
                _FULL2((DIM, 1)), _FULL2((1, 1))]
    return pl.pallas_call(
        body, grid=(nb,), in_specs=in_specs, out_specs=row((R, 1)),
        out_shape=jax.ShapeDtypeStruct((L_SIZE, 1), jnp.float32))


def _prep_edges(idx, pad, fill):
    p = jnp.concatenate([idx, jnp.full((pad,), fill, jnp.int32)])
    return p.reshape(-1, EB)


def kernel(l_edge_index, c_edge_index, l_size, c_size, params):
    E = l_edge_index.shape[0]
    n_iter = -(-E // SWEEP)
    n_iter += n_iter % 2
    e_pad = n_iter * SWEEP
    e_rows = e_pad // EB

    src_l = _prep_edges(l_edge_index, e_pad - E, 0)
    dst_c = _prep_edges(c_edge_index, e_pad - E, C_SIZE)
    src_c = _prep_edges(c_edge_index, e_pad - E, 0)
    dst_l = _prep_edges(l_edge_index, e_pad - E, L_SIZE)

    sc_l2c = _make_sc_aggregate(e_rows, n_iter, L_SIZE, C_SIZE, 2, 32)
    sc_c2l = _make_sc_aggregate(e_rows, n_iter, C_SIZE, L_SIZE, 4, 16)
    lstm_c = _make_lstm_c(2000)
    lstm_l = _make_lstm_l(2000)
    readout = _make_readout(2000)

    p = params
    cc, lc = p['c_cell'], p['l_cell']
    r2 = lambda v: v.reshape(1, -1)
    denom = math.sqrt(DIM)

    l_hidden = jnp.broadcast_to(p['l_init'] / denom, (L_SIZE, DIM))
    c_hidden = jnp.broadcast_to(p['c_init'] / denom, (C_SIZE, DIM))
    l_state = jnp.zeros((L_SIZE, DIM), jnp.float32)
    c_state = jnp.zeros((C_SIZE, DIM), jnp.float32)

    z32 = jnp.zeros((ZROWS, 32), jnp.float32)
    z16 = jnp.zeros((ZROWS, 16), jnp.float32)

    mrow = _init_msg(p['l_init'] / denom, p['l2c_W'][0], r2(p['l2c_b'][0]),
                     p['l2c_W'][1], r2(p['l2c_b'][1]),
                     p['l2c_W'][2], r2(p['l2c_b'][2]))[0]
    msg_l = (jnp.broadcast_to(mrow[0:32], (L_SIZE, 32)),
             jnp.broadcast_to(mrow[32:64], (L_SIZE, 32)))

    for _ in range(N_ROUNDS):
        la0, la1 = sc_l2c(src_l, dst_c, msg_l[0], msg_l[1], z32)
        c_hidden, c_state, cm0, cm1, cm2, cm3 = lstm_c(
            la0, la1, c_hidden, c_state,
            cc['W_ih'], cc['W_hh'], r2(cc['g_ih']), r2(cc['b_ih']),
            r2(cc['g_hh']), r2(cc['b_hh']), r2(cc['g_c']), r2(cc['b_c']),
            p['c2l_W'][0], r2(p['c2l_b'][0]), p['c2l_W'][1], r2(p['c2l_b'][1]),
            p['c2l_W'][2], r2(p['c2l_b'][2]))
        ca0, ca1, ca2, ca3 = sc_c2l(src_c, dst_l, cm0, cm1, cm2, cm3, z16)
        l_hidden, l_state, ml0, ml1 = lstm_l(
            ca0, ca1, ca2, ca3, l_hidden, l_state,
            lc['W_ih'], lc['W_hh'], r2(lc['g_ih']), r2(lc['b_ih']),
            r2(lc['g_hh']), r2(lc['b_hh']), r2(lc['g_c']), r2(lc['b_c']),
            p['l2c_W'][0], r2(p['l2c_b'][0]), p['l2c_W'][1], r2(p['l2c_b'][1]),
            p['l2c_W'][2], r2(p['l2c_b'][2]))
        msg_l = (ml0, ml1)

    probs = readout(l_hidden, p['ro_W'][0], r2(p['ro_b'][0]),
                    p['ro_W'][1], r2(p['ro_b'][1]),
                    p['ro_W'][2], r2(p['ro_b'][2]))
    return probs.reshape(-1, 2)
